# Initial kernel scaffold; baseline (speedup 1.0000x reference)
import jax
import jax.numpy as jnp
from jax.experimental import pallas as pl


def kernel(xyz, new_xyz, feats, W1, g1, be1, rm1, rv1, W2, g2, be2, rm2, rv2, W3, g3, be3, rm3, rv3):
    raise NotImplementedError("write your pallas kernel here")


# speedup vs baseline: 10.8135x; 10.8135x over previous
import functools

import jax
import jax.numpy as jnp
from jax import lax
from jax.experimental import pallas as pl
from jax.experimental.pallas import tpu as pltpu
from jax.experimental.pallas import tpu_sc as plsc

B, N, M, K = 4, 8192, 1024, 32
R2 = 0.2 * 0.2
EPS = 1e-5
BM = B * M
KBM = K * BM

MBLK = 256
NCH = 1024
MB2 = 256
NW = 32
PER_W = KBM // NW
CH = 128


def _prep_body(f_ref, w_ref, g_ref):
    g_ref[...] = jnp.dot(f_ref[...], w_ref[...],
                         preferred_element_type=jnp.float32)


def _cumsum_lanes(x, width):
    lane = lax.broadcasted_iota(jnp.int32, x.shape, 1)
    s = 1
    while s < width:
        shifted = pltpu.roll(x, s, 1)
        x = x + jnp.where(lane >= s, shifted, 0.0)
        s *= 2
    return x


def _select_body(c_ref, xt_ref, w_ref, gidx_ref, ne_ref, cout_ref,
                 acc_ref, cnt_ref):
    cblk = c_ref[...]
    cout_ref[...] = jnp.dot(cblk, w_ref[...],
                            preferred_element_type=jnp.float32)
    cx = cblk[:, 0:1]
    cy = cblk[:, 1:2]
    cz = cblk[:, 2:3]
    acc_ref[...] = jnp.zeros((MBLK, K), jnp.float32)
    cnt_ref[...] = jnp.zeros((MBLK, 1), jnp.float32)

    for ch in range(N // NCH):
        @pl.when(jnp.min(cnt_ref[...]) < K)
        def _chunk():
            xs = xt_ref[0, :, ch * NCH:(ch + 1) * NCH]
            dx = cx - xs[0:1, :]
            dy = cy - xs[1:2, :]
            dz = cz - xs[2:3, :]
            d2 = dx * dx + dy * dy + dz * dz
            m = jnp.where(d2 < R2, 1.0, 0.0)
            rank = cnt_ref[...] + _cumsum_lanes(m, NCH)
            cnt_ref[...] = rank[:, NCH - 1:NCH]
            for k in range(K):
                s = jnp.sum(jnp.where(rank <= k, 1.0, 0.0), axis=1,
                            keepdims=True)
                acc_ref[:, k:k + 1] += s

    acc = acc_ref[...]
    idx0 = acc[:, 0:1]
    fixed = jnp.where(acc >= float(N),
                      jnp.where(idx0 >= float(N), 0.0, idx0), acc)
    batch = pl.program_id(0) * MBLK // M
    gidx_ref[...] = fixed.astype(jnp.int32) + batch * N
    ne_ref[...] = jnp.where(idx0 < float(N), 1.0, 0.0)


def _mlp_body(x_ref, c_ref, ne_ref, w2_ref, w3_ref, s1_ref, t1_ref,
              s2_ref, t2_ref, s3_ref, t3_ref, o_ref):
    k = pl.program_id(1)
    x = x_ref[0]
    h1 = jnp.maximum((x - c_ref[...]) * s1_ref[...] + t1_ref[...], 0.0)
    h2 = jnp.dot(h1, w2_ref[...], preferred_element_type=jnp.float32)
    h2 = jnp.maximum(h2 * s2_ref[...] + t2_ref[...], 0.0)
    h3 = jnp.dot(h2, w3_ref[...], preferred_element_type=jnp.float32)
    h3 = jnp.maximum(h3 * s3_ref[...] + t3_ref[...], 0.0)

    @pl.when(k == 0)
    def _():
        o_ref[...] = h3

    @pl.when(k > 0)
    def _():
        o_ref[...] = jnp.maximum(o_ref[...], h3)

    @pl.when(k == K - 1)
    def _():
        o_ref[...] = o_ref[...] * ne_ref[...]


def _make_gather():
    mesh = plsc.VectorSubcoreMesh(core_axis_name="c", subcore_axis_name="s")

    @functools.partial(
        pl.kernel, mesh=mesh,
        compiler_params=pltpu.CompilerParams(use_tc_tiling_on_sc=False),
        out_type=jax.ShapeDtypeStruct((KBM, 64), jnp.float32),
        scratch_types=[
            pltpu.VMEM((PER_W // CH, CH), jnp.int32),
            pltpu.VMEM((CH, 64), jnp.float32),
            pltpu.SemaphoreType.DMA,
        ],
    )
    def gather_k(table_hbm, idx_hbm, out_hbm, idx_v, rows_v, sem):
        wid = lax.axis_index("s") * 2 + lax.axis_index("c")
        nrow = PER_W // CH
        pltpu.sync_copy(idx_hbm.at[pl.ds(wid * nrow, nrow)], idx_v)
        base = wid * PER_W

        def body(j, carry):
            pltpu.async_copy(table_hbm.at[idx_v.at[j]], rows_v, sem).wait()
            pltpu.sync_copy(rows_v, out_hbm.at[pl.ds(base + j * CH, CH)])
            return carry

        lax.fori_loop(0, nrow, body, 0)

    return gather_k


_gather = _make_gather()


def kernel(xyz, new_xyz, feats, W1, g1, be1, rm1, rv1, W2, g2, be2, rm2,
           rv2, W3, g3, be3, rm3, rv3):
    s1 = g1 * jax.lax.rsqrt(rv1 + EPS)
    t1 = be1 - rm1 * s1
    s2 = g2 * jax.lax.rsqrt(rv2 + EPS)
    t2 = be2 - rm2 * s2
    s3 = g3 * jax.lax.rsqrt(rv3 + EPS)
    t3 = be3 - rm3 * s3

    F = jnp.concatenate([xyz, feats], axis=-1).reshape(B * N, 67)
    G = pl.pallas_call(
        _prep_body,
        grid=(16,),
        in_specs=[
            pl.BlockSpec((B * N // 16, 67), lambda j: (j, 0)),
            pl.BlockSpec((67, 64), lambda j: (0, 0)),
        ],
        out_specs=pl.BlockSpec((B * N // 16, 64), lambda j: (j, 0)),
        out_shape=jax.ShapeDtypeStruct((B * N, 64), jnp.float32),
    )(F, W1)

    cpad = jnp.pad(new_xyz.reshape(BM, 3), ((0, 0), (0, 5)))
    xyzT = jnp.pad(jnp.transpose(xyz, (0, 2, 1)), ((0, 0), (0, 5), (0, 0)))
    w1a = jnp.pad(W1[:3], ((0, 5), (0, 0)))
    gidx, ne, C = pl.pallas_call(
        _select_body,
        grid=(BM // MBLK,),
        in_specs=[
            pl.BlockSpec((MBLK, 8), lambda j: (j, 0)),
            pl.BlockSpec((1, 8, N), lambda j: (j * MBLK // M, 0, 0)),
            pl.BlockSpec((8, 64), lambda j: (0, 0)),
        ],
        out_specs=[
            pl.BlockSpec((MBLK, K), lambda j: (j, 0)),
            pl.BlockSpec((MBLK, 1), lambda j: (j, 0)),
            pl.BlockSpec((MBLK, 64), lambda j: (j, 0)),
        ],
        out_shape=[
            jax.ShapeDtypeStruct((BM, K), jnp.int32),
            jax.ShapeDtypeStruct((BM, 1), jnp.float32),
            jax.ShapeDtypeStruct((BM, 64), jnp.float32),
        ],
        scratch_shapes=[
            pltpu.VMEM((MBLK, K), jnp.float32),
            pltpu.VMEM((MBLK, 1), jnp.float32),
        ],
    )(cpad, xyzT, w1a)

    gidx_t = gidx.T.reshape(KBM // CH, CH)
    rows = _gather(G, gidx_t)

    rows3 = rows.reshape(K, BM, 64)
    r2 = lambda v: v.reshape(1, -1)
    out = pl.pallas_call(
        _mlp_body,
        grid=(BM // MB2, K),
        in_specs=[
            pl.BlockSpec((1, MB2, 64), lambda j, k: (k, j, 0)),
            pl.BlockSpec((MB2, 64), lambda j, k: (j, 0)),
            pl.BlockSpec((MB2, 1), lambda j, k: (j, 0)),
            pl.BlockSpec((64, 64), lambda j, k: (0, 0)),
            pl.BlockSpec((64, 128), lambda j, k: (0, 0)),
            pl.BlockSpec((1, 64), lambda j, k: (0, 0)),
            pl.BlockSpec((1, 64), lambda j, k: (0, 0)),
            pl.BlockSpec((1, 64), lambda j, k: (0, 0)),
            pl.BlockSpec((1, 64), lambda j, k: (0, 0)),
            pl.BlockSpec((1, 128), lambda j, k: (0, 0)),
            pl.BlockSpec((1, 128), lambda j, k: (0, 0)),
        ],
        out_specs=pl.BlockSpec((MB2, 128), lambda j, k: (j, 0)),
        out_shape=jax.ShapeDtypeStruct((BM, 128), jnp.float32),
        compiler_params=pltpu.CompilerParams(
            dimension_semantics=("parallel", "arbitrary")),
    )(rows3, C, ne, W2, W3, r2(s1), r2(t1), r2(s2), r2(t2), r2(s3), r2(t3))

    return (out.reshape(B, M, 128),)


# speedup vs baseline: 12.2496x; 1.1328x over previous
import functools

import jax
import jax.numpy as jnp
from jax import lax
from jax.experimental import pallas as pl
from jax.experimental.pallas import tpu as pltpu
from jax.experimental.pallas import tpu_sc as plsc

B, N, M, K = 4, 8192, 1024, 32
R2 = 0.2 * 0.2
EPS = 1e-5
BM = B * M
KBM = K * BM

MBLK = 256
NCH = 1024
MB2 = 512
NW = 32
PER_W = KBM // NW
CH = 128


def _prep_body(x_ref, f_ref, wa_ref, wb_ref, g_ref):
    g_ref[...] = (
        jnp.dot(x_ref[...], wa_ref[...], preferred_element_type=jnp.float32)
        + jnp.dot(f_ref[...], wb_ref[...], preferred_element_type=jnp.float32))


def _cumsum_lanes(x, width):
    lane = lax.broadcasted_iota(jnp.int32, x.shape, 1)
    s = 1
    while s < width:
        shifted = pltpu.roll(x, s, 1)
        x = x + jnp.where(lane >= s, shifted, 0.0)
        s *= 2
    return x


def _select_body(c_ref, xt_ref, w_ref, gidx_ref, ne_ref, cout_ref,
                 acc_ref, cnt_ref):
    cblk = c_ref[...]
    cout_ref[...] = jnp.dot(cblk, w_ref[...],
                            preferred_element_type=jnp.float32)
    cx = cblk[:, 0:1]
    cy = cblk[:, 1:2]
    cz = cblk[:, 2:3]
    acc_ref[...] = jnp.zeros((MBLK, K), jnp.float32)
    cnt_ref[...] = jnp.zeros((MBLK, 1), jnp.float32)

    for ch in range(N // NCH):
        @pl.when(jnp.min(cnt_ref[...]) < K)
        def _chunk():
            xs = xt_ref[0, :, ch * NCH:(ch + 1) * NCH]
            dx = cx - xs[0:1, :]
            dy = cy - xs[1:2, :]
            dz = cz - xs[2:3, :]
            d2 = dx * dx + dy * dy + dz * dz
            m = jnp.where(d2 < R2, 1.0, 0.0)
            rank = cnt_ref[...] + _cumsum_lanes(m, NCH)
            cnt_ref[...] = rank[:, NCH - 1:NCH]
            for k in range(K):
                s = jnp.sum(jnp.where(rank <= k, 1.0, 0.0), axis=1,
                            keepdims=True)
                acc_ref[:, k:k + 1] += s

    acc = acc_ref[...]
    idx0 = acc[:, 0:1]
    fixed = jnp.where(acc >= float(N),
                      jnp.where(idx0 >= float(N), 0.0, idx0), acc)
    batch = pl.program_id(0) * MBLK // M
    gidx_ref[...] = fixed.astype(jnp.int32) + batch * N
    ne_ref[...] = jnp.where(idx0 < float(N), 1.0, 0.0)


def _mlp_body(x_ref, c_ref, ne_ref, w2_ref, w3_ref, s1_ref, t1_ref,
              s2_ref, t2_ref, s3_ref, t3_ref, o_ref):
    k = pl.program_id(1)
    x = x_ref[0]
    h1 = jnp.maximum((x - c_ref[...]) * s1_ref[...] + t1_ref[...], 0.0)
    h2 = jnp.dot(h1, w2_ref[...], preferred_element_type=jnp.float32)
    h2 = jnp.maximum(h2 * s2_ref[...] + t2_ref[...], 0.0)
    h3 = jnp.dot(h2, w3_ref[...], preferred_element_type=jnp.float32)
    h3 = jnp.maximum(h3 * s3_ref[...] + t3_ref[...], 0.0)

    @pl.when(k == 0)
    def _():
        o_ref[...] = h3

    @pl.when(k > 0)
    def _():
        o_ref[...] = jnp.maximum(o_ref[...], h3)

    @pl.when(k == K - 1)
    def _():
        o_ref[...] = o_ref[...] * ne_ref[...]


def _make_gather():
    mesh = plsc.VectorSubcoreMesh(core_axis_name="c", subcore_axis_name="s")

    @functools.partial(
        pl.kernel, mesh=mesh,
        compiler_params=pltpu.CompilerParams(use_tc_tiling_on_sc=False),
        out_type=jax.ShapeDtypeStruct((KBM, 64), jnp.float32),
        scratch_types=[
            pltpu.VMEM((PER_W // CH, CH), jnp.int32),
            pltpu.VMEM((CH, 64), jnp.float32),
            pltpu.SemaphoreType.DMA,
        ],
    )
    def gather_k(table_hbm, idx_hbm, out_hbm, idx_v, rows_v, sem):
        wid = lax.axis_index("s") * 2 + lax.axis_index("c")
        nrow = PER_W // CH
        pltpu.sync_copy(idx_hbm.at[pl.ds(wid * nrow, nrow)], idx_v)
        base = wid * PER_W

        def body(j, carry):
            pltpu.async_copy(table_hbm.at[idx_v.at[j]], rows_v, sem).wait()
            pltpu.sync_copy(rows_v, out_hbm.at[pl.ds(base + j * CH, CH)])
            return carry

        lax.fori_loop(0, nrow, body, 0)

    return gather_k


_gather = _make_gather()


def kernel(xyz, new_xyz, feats, W1, g1, be1, rm1, rv1, W2, g2, be2, rm2,
           rv2, W3, g3, be3, rm3, rv3):
    s1 = g1 * jax.lax.rsqrt(rv1 + EPS)
    t1 = be1 - rm1 * s1
    s2 = g2 * jax.lax.rsqrt(rv2 + EPS)
    t2 = be2 - rm2 * s2
    s3 = g3 * jax.lax.rsqrt(rv3 + EPS)
    t3 = be3 - rm3 * s3

    G = pl.pallas_call(
        _prep_body,
        grid=(16,),
        in_specs=[
            pl.BlockSpec((B * N // 16, 3), lambda j: (j, 0)),
            pl.BlockSpec((B * N // 16, 64), lambda j: (j, 0)),
            pl.BlockSpec((3, 64), lambda j: (0, 0)),
            pl.BlockSpec((64, 64), lambda j: (0, 0)),
        ],
        out_specs=pl.BlockSpec((B * N // 16, 64), lambda j: (j, 0)),
        out_shape=jax.ShapeDtypeStruct((B * N, 64), jnp.float32),
    )(xyz.reshape(B * N, 3), feats.reshape(B * N, 64), W1[:3], W1[3:])

    cpad = jnp.pad(new_xyz.reshape(BM, 3), ((0, 0), (0, 5)))
    xyzT = jnp.pad(jnp.transpose(xyz, (0, 2, 1)), ((0, 0), (0, 5), (0, 0)))
    w1a = jnp.pad(W1[:3], ((0, 5), (0, 0)))
    gidx, ne, C = pl.pallas_call(
        _select_body,
        grid=(BM // MBLK,),
        in_specs=[
            pl.BlockSpec((MBLK, 8), lambda j: (j, 0)),
            pl.BlockSpec((1, 8, N), lambda j: (j * MBLK // M, 0, 0)),
            pl.BlockSpec((8, 64), lambda j: (0, 0)),
        ],
        out_specs=[
            pl.BlockSpec((MBLK, K), lambda j: (j, 0)),
            pl.BlockSpec((MBLK, 1), lambda j: (j, 0)),
            pl.BlockSpec((MBLK, 64), lambda j: (j, 0)),
        ],
        out_shape=[
            jax.ShapeDtypeStruct((BM, K), jnp.int32),
            jax.ShapeDtypeStruct((BM, 1), jnp.float32),
            jax.ShapeDtypeStruct((BM, 64), jnp.float32),
        ],
        scratch_shapes=[
            pltpu.VMEM((MBLK, K), jnp.float32),
            pltpu.VMEM((MBLK, 1), jnp.float32),
        ],
    )(cpad, xyzT, w1a)

    gidx_t = gidx.T.reshape(KBM // CH, CH)
    rows = _gather(G, gidx_t)

    rows3 = rows.reshape(K, BM, 64)
    r2 = lambda v: v.reshape(1, -1)
    out = pl.pallas_call(
        _mlp_body,
        grid=(BM // MB2, K),
        in_specs=[
            pl.BlockSpec((1, MB2, 64), lambda j, k: (k, j, 0)),
            pl.BlockSpec((MB2, 64), lambda j, k: (j, 0)),
            pl.BlockSpec((MB2, 1), lambda j, k: (j, 0)),
            pl.BlockSpec((64, 64), lambda j, k: (0, 0)),
            pl.BlockSpec((64, 128), lambda j, k: (0, 0)),
            pl.BlockSpec((1, 64), lambda j, k: (0, 0)),
            pl.BlockSpec((1, 64), lambda j, k: (0, 0)),
            pl.BlockSpec((1, 64), lambda j, k: (0, 0)),
            pl.BlockSpec((1, 64), lambda j, k: (0, 0)),
            pl.BlockSpec((1, 128), lambda j, k: (0, 0)),
            pl.BlockSpec((1, 128), lambda j, k: (0, 0)),
        ],
        out_specs=pl.BlockSpec((MB2, 128), lambda j, k: (j, 0)),
        out_shape=jax.ShapeDtypeStruct((BM, 128), jnp.float32),
        compiler_params=pltpu.CompilerParams(
            dimension_semantics=("parallel", "arbitrary")),
    )(rows3, C, ne, W2, W3, r2(s1), r2(t1), r2(s2), r2(t2), r2(s3), r2(t3))

    return (out.reshape(B, M, 128),)


# speedup vs baseline: 18.6983x; 1.5264x over previous
import functools

import jax
import jax.numpy as jnp
from jax import lax
from jax.experimental import pallas as pl
from jax.experimental.pallas import tpu as pltpu
from jax.experimental.pallas import tpu_sc as plsc

B, N, M, K = 4, 8192, 1024, 32
R2 = 0.2 * 0.2
EPS = 1e-5
BM = B * M
KBM = K * BM

MBLK = 256
NCH = 1024
MB2 = 512
NW = 32
PER_W = KBM // NW
CH = 128


def _prep_body(x_ref, f_ref, wa_ref, wb_ref, g_ref):
    g_ref[...] = (
        jnp.dot(x_ref[...], wa_ref[...], preferred_element_type=jnp.float32)
        + jnp.dot(f_ref[...], wb_ref[...], preferred_element_type=jnp.float32))


def _select_body(c_ref, xt_ref, w_ref, ut_ref, gidx_ref, ne_ref, cout_ref,
                 acc_ref, cnt_ref):
    cblk = c_ref[...]
    cout_ref[...] = jnp.dot(cblk, w_ref[...],
                            preferred_element_type=jnp.float32)
    cx = cblk[:, 0:1]
    cy = cblk[:, 1:2]
    cz = cblk[:, 2:3]
    acc_ref[...] = jnp.zeros((MBLK, K), jnp.float32)
    cnt_ref[...] = jnp.zeros((MBLK, 1), jnp.float32)

    for ch in range(N // NCH):
        @pl.when(jnp.min(cnt_ref[...]) < K)
        def _chunk():
            xs = xt_ref[0, :, ch * NCH:(ch + 1) * NCH]
            dx = cx - xs[0:1, :]
            dy = cy - xs[1:2, :]
            dz = cz - xs[2:3, :]
            d2 = dx * dx + dy * dy + dz * dz
            m = jnp.where(d2 < R2, 1.0, 0.0).astype(jnp.bfloat16)
            rank = cnt_ref[...] + jnp.dot(
                m, ut_ref[...], preferred_element_type=jnp.float32)
            cnt_ref[...] = rank[:, NCH - 1:NCH]
            cols = [jnp.sum(jnp.where(rank <= k, 1.0, 0.0), axis=1,
                            keepdims=True) for k in range(K)]
            acc_ref[...] += jnp.concatenate(cols, axis=1)

    acc = acc_ref[...]
    idx0 = acc[:, 0:1]
    fixed = jnp.where(acc >= float(N),
                      jnp.where(idx0 >= float(N), 0.0, idx0), acc)
    batch = pl.program_id(0) * MBLK // M
    gidx_ref[...] = fixed.astype(jnp.int32) + batch * N
    ne_ref[...] = jnp.where(idx0 < float(N), 1.0, 0.0)


def _mlp_body(x_ref, c_ref, ne_ref, w2_ref, w3_ref, s1_ref, t1_ref,
              s2_ref, t2_ref, s3_ref, t3_ref, o_ref):
    k = pl.program_id(1)
    x = x_ref[0]
    h1 = jnp.maximum((x - c_ref[...]) * s1_ref[...] + t1_ref[...], 0.0)
    h2 = jnp.dot(h1, w2_ref[...], preferred_element_type=jnp.float32)
    h2 = jnp.maximum(h2 * s2_ref[...] + t2_ref[...], 0.0)
    h3 = jnp.dot(h2, w3_ref[...], preferred_element_type=jnp.float32)
    h3 = jnp.maximum(h3 * s3_ref[...] + t3_ref[...], 0.0)

    @pl.when(k == 0)
    def _():
        o_ref[...] = h3

    @pl.when(k > 0)
    def _():
        o_ref[...] = jnp.maximum(o_ref[...], h3)

    @pl.when(k == K - 1)
    def _():
        o_ref[...] = o_ref[...] * ne_ref[...]


def _make_gather():
    mesh = plsc.VectorSubcoreMesh(core_axis_name="c", subcore_axis_name="s")

    @functools.partial(
        pl.kernel, mesh=mesh,
        compiler_params=pltpu.CompilerParams(use_tc_tiling_on_sc=False),
        out_type=jax.ShapeDtypeStruct((KBM, 64), jnp.float32),
        scratch_types=[
            pltpu.VMEM((PER_W // CH, CH), jnp.int32),
            pltpu.VMEM((CH, 64), jnp.float32),
            pltpu.SemaphoreType.DMA,
        ],
    )
    def gather_k(table_hbm, idx_hbm, out_hbm, idx_v, rows_v, sem):
        wid = lax.axis_index("s") * 2 + lax.axis_index("c")
        nrow = PER_W // CH
        pltpu.sync_copy(idx_hbm.at[pl.ds(wid * nrow, nrow)], idx_v)
        base = wid * PER_W

        def body(j, carry):
            pltpu.async_copy(table_hbm.at[idx_v.at[j]], rows_v, sem).wait()
            pltpu.sync_copy(rows_v, out_hbm.at[pl.ds(base + j * CH, CH)])
            return carry

        lax.fori_loop(0, nrow, body, 0)

    return gather_k


_gather = _make_gather()


def kernel(xyz, new_xyz, feats, W1, g1, be1, rm1, rv1, W2, g2, be2, rm2,
           rv2, W3, g3, be3, rm3, rv3):
    s1 = g1 * jax.lax.rsqrt(rv1 + EPS)
    t1 = be1 - rm1 * s1
    s2 = g2 * jax.lax.rsqrt(rv2 + EPS)
    t2 = be2 - rm2 * s2
    s3 = g3 * jax.lax.rsqrt(rv3 + EPS)
    t3 = be3 - rm3 * s3

    G = pl.pallas_call(
        _prep_body,
        grid=(16,),
        in_specs=[
            pl.BlockSpec((B * N // 16, 3), lambda j: (j, 0)),
            pl.BlockSpec((B * N // 16, 64), lambda j: (j, 0)),
            pl.BlockSpec((3, 64), lambda j: (0, 0)),
            pl.BlockSpec((64, 64), lambda j: (0, 0)),
        ],
        out_specs=pl.BlockSpec((B * N // 16, 64), lambda j: (j, 0)),
        out_shape=jax.ShapeDtypeStruct((B * N, 64), jnp.float32),
    )(xyz.reshape(B * N, 3), feats.reshape(B * N, 64), W1[:3], W1[3:])

    cpad = jnp.pad(new_xyz.reshape(BM, 3), ((0, 0), (0, 5)))
    xyzT = jnp.pad(jnp.transpose(xyz, (0, 2, 1)), ((0, 0), (0, 5), (0, 0)))
    w1a = jnp.pad(W1[:3], ((0, 5), (0, 0)))
    ut = (lax.broadcasted_iota(jnp.int32, (NCH, NCH), 0)
          <= lax.broadcasted_iota(jnp.int32, (NCH, NCH), 1)
          ).astype(jnp.bfloat16)
    gidx, ne, C = pl.pallas_call(
        _select_body,
        grid=(BM // MBLK,),
        in_specs=[
            pl.BlockSpec((MBLK, 8), lambda j: (j, 0)),
            pl.BlockSpec((1, 8, N), lambda j: (j * MBLK // M, 0, 0)),
            pl.BlockSpec((8, 64), lambda j: (0, 0)),
            pl.BlockSpec((NCH, NCH), lambda j: (0, 0)),
        ],
        out_specs=[
            pl.BlockSpec((MBLK, K), lambda j: (j, 0)),
            pl.BlockSpec((MBLK, 1), lambda j: (j, 0)),
            pl.BlockSpec((MBLK, 64), lambda j: (j, 0)),
        ],
        out_shape=[
            jax.ShapeDtypeStruct((BM, K), jnp.int32),
            jax.ShapeDtypeStruct((BM, 1), jnp.float32),
            jax.ShapeDtypeStruct((BM, 64), jnp.float32),
        ],
        scratch_shapes=[
            pltpu.VMEM((MBLK, K), jnp.float32),
            pltpu.VMEM((MBLK, 1), jnp.float32),
        ],
    )(cpad, xyzT, w1a, ut)

    gidx_t = gidx.T.reshape(KBM // CH, CH)
    rows = _gather(G, gidx_t)

    rows3 = rows.reshape(K, BM, 64)
    r2 = lambda v: v.reshape(1, -1)
    out = pl.pallas_call(
        _mlp_body,
        grid=(BM // MB2, K),
        in_specs=[
            pl.BlockSpec((1, MB2, 64), lambda j, k: (k, j, 0)),
            pl.BlockSpec((MB2, 64), lambda j, k: (j, 0)),
            pl.BlockSpec((MB2, 1), lambda j, k: (j, 0)),
            pl.BlockSpec((64, 64), lambda j, k: (0, 0)),
            pl.BlockSpec((64, 128), lambda j, k: (0, 0)),
            pl.BlockSpec((1, 64), lambda j, k: (0, 0)),
            pl.BlockSpec((1, 64), lambda j, k: (0, 0)),
            pl.BlockSpec((1, 64), lambda j, k: (0, 0)),
            pl.BlockSpec((1, 64), lambda j, k: (0, 0)),
            pl.BlockSpec((1, 128), lambda j, k: (0, 0)),
            pl.BlockSpec((1, 128), lambda j, k: (0, 0)),
        ],
        out_specs=pl.BlockSpec((MB2, 128), lambda j, k: (j, 0)),
        out_shape=jax.ShapeDtypeStruct((BM, 128), jnp.float32),
        compiler_params=pltpu.CompilerParams(
            dimension_semantics=("parallel", "arbitrary")),
    )(rows3, C, ne, W2, W3, r2(s1), r2(t1), r2(s2), r2(t2), r2(s3), r2(t3))

    return (out.reshape(B, M, 128),)


# speedup vs baseline: 19.0193x; 1.0172x over previous
import functools

import jax
import jax.numpy as jnp
from jax import lax
from jax.experimental import pallas as pl
from jax.experimental.pallas import tpu as pltpu
from jax.experimental.pallas import tpu_sc as plsc

B, N, M, K = 4, 8192, 1024, 32
R2 = 0.2 * 0.2
EPS = 1e-5
BM = B * M
KBM = K * BM

MBLK = 256
NCH = 1024
MB2 = 512
NW = 32
PER_W = KBM // NW
CH = 128


def _prep_body(x_ref, f_ref, wa_ref, wb_ref, g_ref):
    g_ref[...] = (
        jnp.dot(x_ref[...], wa_ref[...], preferred_element_type=jnp.float32)
        + jnp.dot(f_ref[...], wb_ref[...], preferred_element_type=jnp.float32))


def _select_body(c_ref, xt_ref, w_ref, ut_ref, gidx_ref, ne_ref, cout_ref,
                 acc_ref, cnt_ref):
    cblk = c_ref[...]
    cout_ref[...] = jnp.dot(cblk, w_ref[...],
                            preferred_element_type=jnp.float32)
    cx = cblk[:, 0:1]
    cy = cblk[:, 1:2]
    cz = cblk[:, 2:3]
    acc_ref[...] = jnp.zeros((MBLK, K), jnp.float32)
    cnt_ref[...] = jnp.zeros((MBLK, 1), jnp.float32)

    for ch in range(N // NCH):
        @pl.when(jnp.min(cnt_ref[...]) < K)
        def _chunk():
            xs = xt_ref[0, :, ch * NCH:(ch + 1) * NCH]
            dx = cx - xs[0:1, :]
            dy = cy - xs[1:2, :]
            dz = cz - xs[2:3, :]
            d2 = dx * dx + dy * dy + dz * dz
            m = jnp.where(d2 < R2, 1.0, 0.0).astype(jnp.bfloat16)
            rank = cnt_ref[...] + jnp.dot(
                m, ut_ref[...], preferred_element_type=jnp.float32)
            cnt_ref[...] = rank[:, NCH - 1:NCH]
            cols = [jnp.sum(jnp.where(rank <= k, 1.0, 0.0), axis=1,
                            keepdims=True) for k in range(K)]
            acc_ref[...] += jnp.concatenate(cols, axis=1)

    acc = acc_ref[...]
    idx0 = acc[:, 0:1]
    fixed = jnp.where(acc >= float(N),
                      jnp.where(idx0 >= float(N), 0.0, idx0), acc)
    batch = pl.program_id(0) * MBLK // M
    gidx_ref[...] = fixed.astype(jnp.int32) + batch * N
    ne_ref[...] = jnp.where(idx0 < float(N), 1.0, 0.0)


def _mlp_body(x_ref, c_ref, ne_ref, w2_ref, w3_ref, s1_ref, t1_ref,
              s2_ref, t2_ref, s3_ref, t3_ref, o_ref):
    k = pl.program_id(1)
    x = x_ref[0]
    h1 = jnp.maximum((x - c_ref[...]) * s1_ref[...] + t1_ref[...], 0.0)
    h2 = jnp.dot(h1.astype(jnp.bfloat16), w2_ref[...],
                 preferred_element_type=jnp.float32)
    h2 = jnp.maximum(h2 * s2_ref[...] + t2_ref[...], 0.0)
    h3 = jnp.dot(h2.astype(jnp.bfloat16), w3_ref[...],
                 preferred_element_type=jnp.float32)
    h3 = jnp.maximum(h3 * s3_ref[...] + t3_ref[...], 0.0)

    @pl.when(k == 0)
    def _():
        o_ref[...] = h3

    @pl.when(k > 0)
    def _():
        o_ref[...] = jnp.maximum(o_ref[...], h3)

    @pl.when(k == K - 1)
    def _():
        o_ref[...] = o_ref[...] * ne_ref[...]


def _make_gather():
    mesh = plsc.VectorSubcoreMesh(core_axis_name="c", subcore_axis_name="s")

    @functools.partial(
        pl.kernel, mesh=mesh,
        compiler_params=pltpu.CompilerParams(use_tc_tiling_on_sc=False),
        out_type=jax.ShapeDtypeStruct((KBM, 64), jnp.float32),
        scratch_types=[
            pltpu.VMEM((PER_W // CH, CH), jnp.int32),
            pltpu.VMEM((CH, 64), jnp.float32),
            pltpu.VMEM((CH, 64), jnp.float32),
            pltpu.SemaphoreType.DMA,
            pltpu.SemaphoreType.DMA,
        ],
    )
    def gather_k(table_hbm, idx_hbm, out_hbm, idx_v, rows_v0, rows_v1,
                 sem0, sem1):
        wid = lax.axis_index("s") * 2 + lax.axis_index("c")
        nrow = PER_W // CH
        pltpu.sync_copy(idx_hbm.at[pl.ds(wid * nrow, nrow)], idx_v)
        base = wid * PER_W

        cp0 = pltpu.async_copy(table_hbm.at[idx_v.at[0]], rows_v0, sem0)

        def body(p, carry):
            j0 = 2 * p
            pltpu.async_copy(table_hbm.at[idx_v.at[j0 + 1]], rows_v1, sem1)
            pltpu.make_async_copy(table_hbm.at[idx_v.at[j0]], rows_v0,
                                  sem0).wait()
            pltpu.sync_copy(rows_v0, out_hbm.at[pl.ds(base + j0 * CH, CH)])

            @pl.when(p < nrow // 2 - 1)
            def _():
                pltpu.async_copy(table_hbm.at[idx_v.at[j0 + 2]], rows_v0,
                                 sem0)

            pltpu.make_async_copy(table_hbm.at[idx_v.at[j0 + 1]], rows_v1,
                                  sem1).wait()
            pltpu.sync_copy(rows_v1,
                            out_hbm.at[pl.ds(base + (j0 + 1) * CH, CH)])
            return carry

        lax.fori_loop(0, nrow // 2, body, 0)

    return gather_k


_gather = _make_gather()


def kernel(xyz, new_xyz, feats, W1, g1, be1, rm1, rv1, W2, g2, be2, rm2,
           rv2, W3, g3, be3, rm3, rv3):
    s1 = g1 * jax.lax.rsqrt(rv1 + EPS)
    t1 = be1 - rm1 * s1
    s2 = g2 * jax.lax.rsqrt(rv2 + EPS)
    t2 = be2 - rm2 * s2
    s3 = g3 * jax.lax.rsqrt(rv3 + EPS)
    t3 = be3 - rm3 * s3

    G = pl.pallas_call(
        _prep_body,
        grid=(16,),
        in_specs=[
            pl.BlockSpec((B * N // 16, 3), lambda j: (j, 0)),
            pl.BlockSpec((B * N // 16, 64), lambda j: (j, 0)),
            pl.BlockSpec((3, 64), lambda j: (0, 0)),
            pl.BlockSpec((64, 64), lambda j: (0, 0)),
        ],
        out_specs=pl.BlockSpec((B * N // 16, 64), lambda j: (j, 0)),
        out_shape=jax.ShapeDtypeStruct((B * N, 64), jnp.float32),
    )(xyz.reshape(B * N, 3), feats.reshape(B * N, 64), W1[:3], W1[3:])

    cpad = jnp.pad(new_xyz.reshape(BM, 3), ((0, 0), (0, 5)))
    xyzT = jnp.pad(jnp.transpose(xyz, (0, 2, 1)), ((0, 0), (0, 5), (0, 0)))
    w1a = jnp.pad(W1[:3], ((0, 5), (0, 0)))
    ut = (lax.broadcasted_iota(jnp.int32, (NCH, NCH), 0)
          <= lax.broadcasted_iota(jnp.int32, (NCH, NCH), 1)
          ).astype(jnp.bfloat16)
    gidx, ne, C = pl.pallas_call(
        _select_body,
        grid=(BM // MBLK,),
        in_specs=[
            pl.BlockSpec((MBLK, 8), lambda j: (j, 0)),
            pl.BlockSpec((1, 8, N), lambda j: (j * MBLK // M, 0, 0)),
            pl.BlockSpec((8, 64), lambda j: (0, 0)),
            pl.BlockSpec((NCH, NCH), lambda j: (0, 0)),
        ],
        out_specs=[
            pl.BlockSpec((MBLK, K), lambda j: (j, 0)),
            pl.BlockSpec((MBLK, 1), lambda j: (j, 0)),
            pl.BlockSpec((MBLK, 64), lambda j: (j, 0)),
        ],
        out_shape=[
            jax.ShapeDtypeStruct((BM, K), jnp.int32),
            jax.ShapeDtypeStruct((BM, 1), jnp.float32),
            jax.ShapeDtypeStruct((BM, 64), jnp.float32),
        ],
        scratch_shapes=[
            pltpu.VMEM((MBLK, K), jnp.float32),
            pltpu.VMEM((MBLK, 1), jnp.float32),
        ],
    )(cpad, xyzT, w1a, ut)

    gidx_t = gidx.T.reshape(KBM // CH, CH)
    rows = _gather(G, gidx_t)

    rows3 = rows.reshape(K, BM, 64)
    r2 = lambda v: v.reshape(1, -1)
    out = pl.pallas_call(
        _mlp_body,
        grid=(BM // MB2, K),
        in_specs=[
            pl.BlockSpec((1, MB2, 64), lambda j, k: (k, j, 0)),
            pl.BlockSpec((MB2, 64), lambda j, k: (j, 0)),
            pl.BlockSpec((MB2, 1), lambda j, k: (j, 0)),
            pl.BlockSpec((64, 64), lambda j, k: (0, 0)),
            pl.BlockSpec((64, 128), lambda j, k: (0, 0)),
            pl.BlockSpec((1, 64), lambda j, k: (0, 0)),
            pl.BlockSpec((1, 64), lambda j, k: (0, 0)),
            pl.BlockSpec((1, 64), lambda j, k: (0, 0)),
            pl.BlockSpec((1, 64), lambda j, k: (0, 0)),
            pl.BlockSpec((1, 128), lambda j, k: (0, 0)),
            pl.BlockSpec((1, 128), lambda j, k: (0, 0)),
        ],
        out_specs=pl.BlockSpec((MB2, 128), lambda j, k: (j, 0)),
        out_shape=jax.ShapeDtypeStruct((BM, 128), jnp.float32),
        compiler_params=pltpu.CompilerParams(
            dimension_semantics=("parallel", "arbitrary")),
    )(rows3, C, ne, W2.astype(jnp.bfloat16), W3.astype(jnp.bfloat16),
      r2(s1), r2(t1), r2(s2), r2(t2), r2(s3), r2(t3))

    return (out.reshape(B, M, 128),)


# speedup vs baseline: 21.4819x; 1.1295x over previous
import functools

import jax
import jax.numpy as jnp
from jax import lax
from jax.experimental import pallas as pl
from jax.experimental.pallas import tpu as pltpu
from jax.experimental.pallas import tpu_sc as plsc

B, N, M, K = 4, 8192, 1024, 32
R2 = 0.2 * 0.2
EPS = 1e-5
BM = B * M
KBM = K * BM

MBLK = 256
NCH = 1024
MB2 = 1024
NW = 32
PER_W = KBM // NW
CH = 128


def _prep_body(x_ref, f_ref, wa_ref, wb_ref, t1_ref, g_ref):
    g_ref[...] = (
        jnp.dot(x_ref[...], wa_ref[...], preferred_element_type=jnp.float32)
        + jnp.dot(f_ref[...], wb_ref[...], preferred_element_type=jnp.float32)
        + t1_ref[...])


def _select_body(c_ref, xt_ref, w_ref, ut_ref, gidx_ref, ne_ref, cout_ref,
                 acc_ref, cnt_ref):
    cblk = c_ref[...]
    cout_ref[...] = jnp.dot(cblk, w_ref[...],
                            preferred_element_type=jnp.float32)
    cx = cblk[:, 0:1]
    cy = cblk[:, 1:2]
    cz = cblk[:, 2:3]
    acc_ref[...] = jnp.zeros((MBLK, K), jnp.float32)
    cnt_ref[...] = jnp.zeros((MBLK, 1), jnp.float32)

    for ch in range(N // NCH):
        @pl.when(jnp.min(cnt_ref[...]) < K)
        def _chunk():
            xs = xt_ref[0, :, ch * NCH:(ch + 1) * NCH]
            dx = cx - xs[0:1, :]
            dy = cy - xs[1:2, :]
            dz = cz - xs[2:3, :]
            d2 = dx * dx + dy * dy + dz * dz
            m = jnp.where(d2 < R2, 1.0, 0.0).astype(jnp.bfloat16)
            rank = cnt_ref[...] + jnp.dot(
                m, ut_ref[...], preferred_element_type=jnp.float32)
            cnt_ref[...] = rank[:, NCH - 1:NCH]
            cols = [jnp.sum(jnp.where(rank <= k, 1.0, 0.0), axis=1,
                            keepdims=True) for k in range(K)]
            acc_ref[...] += jnp.concatenate(cols, axis=1)

    acc = acc_ref[...]
    idx0 = acc[:, 0:1]
    fixed = jnp.where(acc >= float(N),
                      jnp.where(idx0 >= float(N), 0.0, idx0), acc)
    batch = pl.program_id(0) * MBLK // M
    gidx_ref[...] = fixed.astype(jnp.int32) + batch * N
    ne_ref[...] = jnp.where(idx0 < float(N), 1.0, 0.0)


def _mlp_body(x_ref, c_ref, ne_ref, w2_ref, w3_ref, t2_ref, t3_ref, o_ref):
    k = pl.program_id(1)
    x = x_ref[0]
    h1 = jnp.maximum(x - c_ref[...], 0.0)
    h2 = jnp.dot(h1.astype(jnp.bfloat16), w2_ref[...],
                 preferred_element_type=jnp.float32)
    h2 = jnp.maximum(h2 + t2_ref[...], 0.0)
    h3 = jnp.dot(h2.astype(jnp.bfloat16), w3_ref[...],
                 preferred_element_type=jnp.float32)
    h3 = jnp.maximum(h3 + t3_ref[...], 0.0)

    @pl.when(k == 0)
    def _():
        o_ref[...] = h3

    @pl.when(k > 0)
    def _():
        o_ref[...] = jnp.maximum(o_ref[...], h3)

    @pl.when(k == K - 1)
    def _():
        o_ref[...] = o_ref[...] * ne_ref[...]


def _make_gather():
    mesh = plsc.VectorSubcoreMesh(core_axis_name="c", subcore_axis_name="s")

    @functools.partial(
        pl.kernel, mesh=mesh,
        compiler_params=pltpu.CompilerParams(use_tc_tiling_on_sc=False),
        out_type=jax.ShapeDtypeStruct((KBM, 64), jnp.float32),
        scratch_types=[
            pltpu.VMEM((PER_W // CH, CH), jnp.int32),
            pltpu.VMEM((CH, 64), jnp.float32),
            pltpu.VMEM((CH, 64), jnp.float32),
            pltpu.SemaphoreType.DMA,
            pltpu.SemaphoreType.DMA,
        ],
    )
    def gather_k(table_hbm, idx_hbm, out_hbm, idx_v, rows_v0, rows_v1,
                 sem0, sem1):
        wid = lax.axis_index("s") * 2 + lax.axis_index("c")
        nrow = PER_W // CH
        pltpu.sync_copy(idx_hbm.at[pl.ds(wid * nrow, nrow)], idx_v)
        base = wid * PER_W

        cp0 = pltpu.async_copy(table_hbm.at[idx_v.at[0]], rows_v0, sem0)

        def body(p, carry):
            j0 = 2 * p
            pltpu.async_copy(table_hbm.at[idx_v.at[j0 + 1]], rows_v1, sem1)
            pltpu.make_async_copy(table_hbm.at[idx_v.at[j0]], rows_v0,
                                  sem0).wait()
            pltpu.sync_copy(rows_v0, out_hbm.at[pl.ds(base + j0 * CH, CH)])

            @pl.when(p < nrow // 2 - 1)
            def _():
                pltpu.async_copy(table_hbm.at[idx_v.at[j0 + 2]], rows_v0,
                                 sem0)

            pltpu.make_async_copy(table_hbm.at[idx_v.at[j0 + 1]], rows_v1,
                                  sem1).wait()
            pltpu.sync_copy(rows_v1,
                            out_hbm.at[pl.ds(base + (j0 + 1) * CH, CH)])
            return carry

        lax.fori_loop(0, nrow // 2, body, 0)

    return gather_k


_gather = _make_gather()


def kernel(xyz, new_xyz, feats, W1, g1, be1, rm1, rv1, W2, g2, be2, rm2,
           rv2, W3, g3, be3, rm3, rv3):
    s1 = g1 * jax.lax.rsqrt(rv1 + EPS)
    t1 = be1 - rm1 * s1
    s2 = g2 * jax.lax.rsqrt(rv2 + EPS)
    t2 = be2 - rm2 * s2
    s3 = g3 * jax.lax.rsqrt(rv3 + EPS)
    t3 = be3 - rm3 * s3

    W1f = W1 * s1[None, :]
    G = pl.pallas_call(
        _prep_body,
        grid=(16,),
        in_specs=[
            pl.BlockSpec((B * N // 16, 3), lambda j: (j, 0)),
            pl.BlockSpec((B * N // 16, 64), lambda j: (j, 0)),
            pl.BlockSpec((3, 64), lambda j: (0, 0)),
            pl.BlockSpec((64, 64), lambda j: (0, 0)),
            pl.BlockSpec((1, 64), lambda j: (0, 0)),
        ],
        out_specs=pl.BlockSpec((B * N // 16, 64), lambda j: (j, 0)),
        out_shape=jax.ShapeDtypeStruct((B * N, 64), jnp.float32),
    )(xyz.reshape(B * N, 3), feats.reshape(B * N, 64), W1f[:3], W1f[3:],
      t1.reshape(1, 64))

    cpad = jnp.pad(new_xyz.reshape(BM, 3), ((0, 0), (0, 5)))
    xyzT = jnp.pad(jnp.transpose(xyz, (0, 2, 1)), ((0, 0), (0, 5), (0, 0)))
    w1a = jnp.pad(W1f[:3], ((0, 5), (0, 0)))
    ut = (lax.broadcasted_iota(jnp.int32, (NCH, NCH), 0)
          <= lax.broadcasted_iota(jnp.int32, (NCH, NCH), 1)
          ).astype(jnp.bfloat16)
    gidx, ne, C = pl.pallas_call(
        _select_body,
        grid=(BM // MBLK,),
        in_specs=[
            pl.BlockSpec((MBLK, 8), lambda j: (j, 0)),
            pl.BlockSpec((1, 8, N), lambda j: (j * MBLK // M, 0, 0)),
            pl.BlockSpec((8, 64), lambda j: (0, 0)),
            pl.BlockSpec((NCH, NCH), lambda j: (0, 0)),
        ],
        out_specs=[
            pl.BlockSpec((MBLK, K), lambda j: (j, 0)),
            pl.BlockSpec((MBLK, 1), lambda j: (j, 0)),
            pl.BlockSpec((MBLK, 64), lambda j: (j, 0)),
        ],
        out_shape=[
            jax.ShapeDtypeStruct((BM, K), jnp.int32),
            jax.ShapeDtypeStruct((BM, 1), jnp.float32),
            jax.ShapeDtypeStruct((BM, 64), jnp.float32),
        ],
        scratch_shapes=[
            pltpu.VMEM((MBLK, K), jnp.float32),
            pltpu.VMEM((MBLK, 1), jnp.float32),
        ],
    )(cpad, xyzT, w1a, ut)

    gidx_t = gidx.T.reshape(KBM // CH, CH)
    rows = _gather(G, gidx_t)

    rows3 = rows.reshape(K, BM, 64)
    r2 = lambda v: v.reshape(1, -1)
    out = pl.pallas_call(
        _mlp_body,
        grid=(BM // MB2, K),
        in_specs=[
            pl.BlockSpec((1, MB2, 64), lambda j, k: (k, j, 0)),
            pl.BlockSpec((MB2, 64), lambda j, k: (j, 0)),
            pl.BlockSpec((MB2, 1), lambda j, k: (j, 0)),
            pl.BlockSpec((64, 64), lambda j, k: (0, 0)),
            pl.BlockSpec((64, 128), lambda j, k: (0, 0)),
            pl.BlockSpec((1, 64), lambda j, k: (0, 0)),
            pl.BlockSpec((1, 128), lambda j, k: (0, 0)),
        ],
        out_specs=pl.BlockSpec((MB2, 128), lambda j, k: (j, 0)),
        out_shape=jax.ShapeDtypeStruct((BM, 128), jnp.float32),
        compiler_params=pltpu.CompilerParams(
            dimension_semantics=("parallel", "arbitrary")),
    )(rows3, C, ne, (W2 * s2[None, :]).astype(jnp.bfloat16),
      (W3 * s3[None, :]).astype(jnp.bfloat16), r2(t2), r2(t3))

    return (out.reshape(B, M, 128),)
